# Initial kernel scaffold; baseline (speedup 1.0000x reference)
import functools

import jax
import jax.numpy as jnp
from jax import lax
from jax.experimental import pallas as pl
from jax.experimental.pallas import tpu as pltpu
from jax.experimental.pallas import tpu_sc as plsc

_LANES = 16


def _dense_proj(x, WkT, bk2, WqT, bq2, WvT, bv2, WsT, b2):
    n, d_in = x.shape
    d_out = WkT.shape[1]
    bn = 1000
    grid = (n // bn,)

    def body(x_ref, wk, bkr, wq, bqr, wv, bvr, ws, br, k_r, q_r, v_r, o_r):
        xb = x_ref[...]
        k_r[...] = jnp.dot(xb, wk[...], preferred_element_type=jnp.float32) + bkr[...]
        q_r[...] = jnp.dot(xb, wq[...], preferred_element_type=jnp.float32) + bqr[...]
        v_r[...] = jnp.dot(xb, wv[...], preferred_element_type=jnp.float32) + bvr[...]
        o_r[...] = jnp.dot(xb, ws[...], preferred_element_type=jnp.float32) + br[...]

    row_spec = pl.BlockSpec((bn, d_in), lambda i: (i, 0))
    w_spec = pl.BlockSpec((d_in, d_out), lambda i: (0, 0))
    b_spec = pl.BlockSpec((1, d_out), lambda i: (0, 0))
    out_sds = jax.ShapeDtypeStruct((n, d_out), jnp.float32)
    return pl.pallas_call(
        body,
        grid=grid,
        in_specs=[row_spec, w_spec, b_spec, w_spec, b_spec, w_spec, b_spec,
                  w_spec, b_spec],
        out_specs=[pl.BlockSpec((bn, d_out), lambda i: (i, 0))] * 4,
        out_shape=[out_sds] * 4,
    )(x, WkT, bk2, WqT, bq2, WvT, bv2, WsT, b2)


def _edge_aggregate(src, dst, k, q, v, zeros_nd):
    e = src.shape[0]
    n, d = k.shape
    groups = d // _LANES
    mesh = plsc.VectorSubcoreMesh(core_axis_name="c", subcore_axis_name="s")
    n_tiles = 32
    e_per_tile = e // n_tiles
    ch = 80
    n_ch = e_per_tile // ch
    rows_per_sub = n // 16

    @functools.partial(
        pl.kernel,
        out_type=jax.ShapeDtypeStruct((2, n, d), jnp.float32),
        mesh=mesh,
        scratch_types=[
            pltpu.VMEM((ch,), jnp.int32),
            pltpu.VMEM((ch,), jnp.int32),
            pltpu.VMEM((ch, d), jnp.float32),
            pltpu.VMEM((ch, d), jnp.float32),
            pltpu.VMEM((ch, d), jnp.float32),
            pltpu.VMEM_SHARED((n, d), jnp.float32),
            pltpu.SemaphoreType.DMA,
        ],
    )
    def edge_kernel(src_h, dst_h, k_h, q_h, v_h, zeros_h, out_h,
                    src_v, dst_v, kd_v, qs_v, vs_v, agg_sh, sem):
        c = lax.axis_index("c")
        s = lax.axis_index("s")
        tid = s * 2 + c

        pltpu.sync_copy(zeros_h.at[pl.ds(s * rows_per_sub, rows_per_sub)],
                        agg_sh.at[pl.ds(s * rows_per_sub, rows_per_sub)])
        plsc.subcore_barrier()

        ebase = tid * e_per_tile

        def chunk_body(ci, carry):
            base = ebase + ci * ch
            pltpu.sync_copy(src_h.at[pl.ds(base, ch)], src_v)
            pltpu.sync_copy(dst_h.at[pl.ds(base, ch)], dst_v)
            cp1 = pltpu.async_copy(k_h.at[dst_v], kd_v, sem)
            cp2 = pltpu.async_copy(q_h.at[src_v], qs_v, sem)
            cp3 = pltpu.async_copy(v_h.at[src_v], vs_v, sem)
            cp1.wait()
            cp2.wait()
            cp3.wait()

            def edge_body(i, carry2):
                for g in range(groups):
                    sl = pl.ds(g * _LANES, _LANES)
                    z = kd_v[i, sl] + qs_v[i, sl]
                    gate = 1.0 / (1.0 + jnp.exp(-z))
                    vs_v[i, sl] = gate * vs_v[i, sl]
                return carry2

            lax.fori_loop(0, ch, edge_body, 0)
            pltpu.sync_copy(vs_v, agg_sh.at[dst_v], add=True)
            return carry

        lax.fori_loop(0, n_ch, chunk_body, 0)
        plsc.subcore_barrier()

        pltpu.sync_copy(agg_sh.at[pl.ds(s * rows_per_sub, rows_per_sub)],
                        out_h.at[c, pl.ds(s * rows_per_sub, rows_per_sub)])

    return edge_kernel(src, dst, k, q, v, zeros_nd)


def _combine(skip, aggs):
    n, d = skip.shape
    bn = 1000
    grid = (n // bn,)

    def body(s_ref, a_ref, o_ref):
        o_ref[...] = s_ref[...] + a_ref[0] + a_ref[1]

    return pl.pallas_call(
        body,
        grid=grid,
        in_specs=[pl.BlockSpec((bn, d), lambda i: (i, 0)),
                  pl.BlockSpec((2, bn, d), lambda i: (0, i, 0))],
        out_specs=pl.BlockSpec((bn, d), lambda i: (i, 0)),
        out_shape=jax.ShapeDtypeStruct((n, d), jnp.float32),
    )(skip, aggs)


def kernel(x, edge_index, Wk, bk, Wq, bq, Wv, bv, Ws, b):
    n, d_in = x.shape
    d_out = Wk.shape[0]
    k, q, v, skip = _dense_proj(
        x,
        Wk.T, bk.reshape(1, d_out),
        Wq.T, bq.reshape(1, d_out),
        Wv.T, bv.reshape(1, d_out),
        Ws.T, b.reshape(1, d_out),
    )
    src = edge_index[0]
    dst = edge_index[1]
    zeros_nd = jnp.zeros((n, d_out), jnp.float32)
    aggs = _edge_aggregate(src, dst, k, q, v, zeros_nd)
    return _combine(skip, aggs)


# speedup vs baseline: 5.3651x; 5.3651x over previous
import functools

import jax
import jax.numpy as jnp
from jax import lax
from jax.experimental import pallas as pl
from jax.experimental.pallas import tpu as pltpu
from jax.experimental.pallas import tpu_sc as plsc

_LANES = 16


def _dense_proj(x, WkT, bk2, WqT, bq2, WvT, bv2, WsT, b2):
    n, d_in = x.shape
    d_out = WkT.shape[1]
    bn = 1000
    grid = (n // bn,)

    def body(x_ref, wk, bkr, wq, bqr, wv, bvr, ws, br, k_r, q_r, v_r, o_r):
        xb = x_ref[...]
        k_r[...] = jnp.dot(xb, wk[...], preferred_element_type=jnp.float32) + bkr[...]
        q_r[...] = jnp.dot(xb, wq[...], preferred_element_type=jnp.float32) + bqr[...]
        v_r[...] = jnp.dot(xb, wv[...], preferred_element_type=jnp.float32) + bvr[...]
        o_r[...] = jnp.dot(xb, ws[...], preferred_element_type=jnp.float32) + br[...]

    row_spec = pl.BlockSpec((bn, d_in), lambda i: (i, 0))
    w_spec = pl.BlockSpec((d_in, d_out), lambda i: (0, 0))
    b_spec = pl.BlockSpec((1, d_out), lambda i: (0, 0))
    out_sds = jax.ShapeDtypeStruct((n, d_out), jnp.float32)
    return pl.pallas_call(
        body,
        grid=grid,
        in_specs=[row_spec, w_spec, b_spec, w_spec, b_spec, w_spec, b_spec,
                  w_spec, b_spec],
        out_specs=[pl.BlockSpec((bn, d_out), lambda i: (i, 0))] * 4,
        out_shape=[out_sds] * 4,
    )(x, WkT, bk2, WqT, bq2, WvT, bv2, WsT, b2)


def _edge_aggregate(src, dst, k, q, v, zeros_nd):
    e = src.shape[0]
    n, d = k.shape
    n_pad = zeros_nd.shape[0]
    groups = d // _LANES
    mesh = plsc.VectorSubcoreMesh(core_axis_name="c", subcore_axis_name="s")
    n_tiles = 32
    e_per_tile = e // n_tiles
    ch = 80
    n_ch = e_per_tile // ch
    rows_per_sub = n_pad // 16

    @functools.partial(
        pl.kernel,
        out_type=jax.ShapeDtypeStruct((2, n_pad, d), jnp.float32),
        mesh=mesh,
        scratch_types=[
            pltpu.VMEM((ch,), jnp.int32),
            pltpu.VMEM((ch,), jnp.int32),
            pltpu.VMEM((ch, d), jnp.float32),
            pltpu.VMEM((ch, d), jnp.float32),
            pltpu.VMEM((ch, d), jnp.float32),
            pltpu.VMEM_SHARED((n_pad, d), jnp.float32),
            pltpu.SemaphoreType.DMA,
        ],
    )
    def edge_kernel(src_h, dst_h, k_h, q_h, v_h, zeros_h, out_h,
                    src_v, dst_v, kd_v, qs_v, vs_v, agg_sh, sem):
        c = lax.axis_index("c")
        s = lax.axis_index("s")
        tid = s * 2 + c

        pltpu.sync_copy(zeros_h.at[pl.ds(s * rows_per_sub, rows_per_sub)],
                        agg_sh.at[pl.ds(s * rows_per_sub, rows_per_sub)])
        plsc.subcore_barrier()

        ebase = tid * e_per_tile

        def chunk_body(ci, carry):
            base = ebase + ci * ch
            pltpu.sync_copy(src_h.at[pl.ds(base, ch)], src_v)
            pltpu.sync_copy(dst_h.at[pl.ds(base, ch)], dst_v)
            cp1 = pltpu.async_copy(k_h.at[dst_v], kd_v, sem)
            cp2 = pltpu.async_copy(q_h.at[src_v], qs_v, sem)
            cp3 = pltpu.async_copy(v_h.at[src_v], vs_v, sem)
            cp1.wait()
            cp2.wait()
            cp3.wait()

            def edge_body(i, carry2):
                for g in range(groups):
                    sl = pl.ds(g * _LANES, _LANES)
                    z = kd_v[i, sl] + qs_v[i, sl]
                    gate = 1.0 / (1.0 + jnp.exp(-z))
                    vs_v[i, sl] = gate * vs_v[i, sl]
                return carry2

            lax.fori_loop(0, ch, edge_body, 0)
            pltpu.sync_copy(vs_v, agg_sh.at[dst_v], add=True)
            return carry

        lax.fori_loop(0, n_ch, chunk_body, 0)
        plsc.subcore_barrier()

        pltpu.sync_copy(agg_sh.at[pl.ds(s * rows_per_sub, rows_per_sub)],
                        out_h.at[c, pl.ds(s * rows_per_sub, rows_per_sub)])

    return edge_kernel(src, dst, k, q, v, zeros_nd)


def _combine(skip, aggs):
    n, d = skip.shape
    bn = 1000
    grid = (n // bn,)

    def body(s_ref, a_ref, o_ref):
        o_ref[...] = s_ref[...] + a_ref[0] + a_ref[1]

    return pl.pallas_call(
        body,
        grid=grid,
        in_specs=[pl.BlockSpec((bn, d), lambda i: (i, 0)),
                  pl.BlockSpec((2, bn, d), lambda i: (0, i, 0))],
        out_specs=pl.BlockSpec((bn, d), lambda i: (i, 0)),
        out_shape=jax.ShapeDtypeStruct((n, d), jnp.float32),
    )(skip, aggs)


def kernel(x, edge_index, Wk, bk, Wq, bq, Wv, bv, Ws, b):
    n, d_in = x.shape
    d_out = Wk.shape[0]
    k, q, v, skip = _dense_proj(
        x,
        Wk.T, bk.reshape(1, d_out),
        Wq.T, bq.reshape(1, d_out),
        Wv.T, bv.reshape(1, d_out),
        Ws.T, b.reshape(1, d_out),
    )
    src = edge_index[0]
    dst = edge_index[1]
    n_pad = ((n + 127) // 128) * 128
    zeros_nd = jnp.zeros((n_pad, d_out), jnp.float32)
    aggs = _edge_aggregate(src, dst, k, q, v, zeros_nd)
    return _combine(skip, aggs)
